# baseline (speedup 1.0000x reference)
import functools

import jax
import jax.numpy as jnp
from jax import lax
from jax.experimental import pallas as pl
from jax.experimental.pallas import tpu as pltpu
from jax.experimental.pallas import tpu_sc as plsc

N = 10000
C_IN = 256
C_OUT = 256
C_HALF = 128
K = 27
BN_EPS = 1e-5

E = N * K
NUM_TILES = 16
CHUNK = 128
NCHUNK = 132
NPAIR = NCHUNK // 2
E_TILE = NCHUNK * CHUNK
E_PAD = NUM_TILES * E_TILE
DUMP = N
ACC_ROWS = 10008
STRIPE = 632
LAST_STRIPE = ACC_ROWS - (NUM_TILES - 1) * STRIPE


def _matmul_tc(data, weight):
    def body(d_ref, w_ref, l_ref, r_ref):
        res = jnp.dot(d_ref[...], w_ref[0],
                      preferred_element_type=jnp.float32).astype(jnp.bfloat16)
        l_ref[...] = res[:, :C_HALF]
        r_ref[...] = res[:, C_HALF:]

    return pl.pallas_call(
        body,
        grid=(K,),
        in_specs=[
            pl.BlockSpec((N, C_IN), lambda k: (0, 0)),
            pl.BlockSpec((1, C_IN, C_OUT), lambda k: (k, 0, 0)),
        ],
        out_specs=[
            pl.BlockSpec((N, C_HALF), lambda k: (k, 0)),
            pl.BlockSpec((N, C_HALF), lambda k: (k, 0)),
        ],
        out_shape=[
            jax.ShapeDtypeStruct((E_PAD, C_HALF), jnp.bfloat16),
            jax.ShapeDtypeStruct((E_PAD, C_HALF), jnp.bfloat16),
        ],
    )(data.astype(jnp.bfloat16), weight.astype(jnp.bfloat16))


def _scatter_sc(contrib_l, contrib_r, idx, zeros):
    mesh = plsc.VectorSubcoreMesh(core_axis_name="c", subcore_axis_name="s")

    @functools.partial(
        pl.kernel,
        out_type=jax.ShapeDtypeStruct((2, 2, ACC_ROWS, C_HALF), jnp.bfloat16),
        mesh=mesh,
        compiler_params=pltpu.CompilerParams(use_tc_tiling_on_sc=False),
        scratch_types=[
            pltpu.VMEM((CHUNK,), jnp.int32),
            pltpu.VMEM((CHUNK,), jnp.int32),
            pltpu.VMEM((CHUNK, C_HALF), jnp.bfloat16),
            pltpu.VMEM((CHUNK, C_HALF), jnp.bfloat16),
            pltpu.VMEM_SHARED((ACC_ROWS, C_HALF), jnp.bfloat16),
            pltpu.VMEM_SHARED((ACC_ROWS, C_HALF), jnp.bfloat16),
            pltpu.SemaphoreType.DMA,
            pltpu.SemaphoreType.DMA,
        ],
    )
    def body(cl_hbm, cr_hbm, idx_hbm, zeros_hbm, out_hbm,
             cidx0, cidx1, buf0, buf1, acc_a, acc_b, sem0, sem1):
        c = lax.axis_index("c")
        s = lax.axis_index("s")
        base = s * E_TILE

        @pl.when(s < NUM_TILES - 1)
        def _():
            pltpu.sync_copy(zeros_hbm, acc_a.at[pl.ds(s * STRIPE, STRIPE)])
            pltpu.sync_copy(zeros_hbm, acc_b.at[pl.ds(s * STRIPE, STRIPE)])

        @pl.when(s == NUM_TILES - 1)
        def _():
            pltpu.sync_copy(zeros_hbm.at[pl.ds(0, LAST_STRIPE)],
                            acc_a.at[pl.ds(s * STRIPE, LAST_STRIPE)])
            pltpu.sync_copy(zeros_hbm.at[pl.ds(0, LAST_STRIPE)],
                            acc_b.at[pl.ds(s * STRIPE, LAST_STRIPE)])

        plsc.subcore_barrier()

        def run(src_hbm):
            def load(it, cidx, buf, sem):
                pltpu.async_copy(idx_hbm.at[s, it], cidx, sem)
                pltpu.async_copy(
                    src_hbm.at[pl.ds(base + it * CHUNK, CHUNK)], buf, sem)

            def wait(cidx, buf, sem):
                pltpu.make_async_copy(idx_hbm.at[s, 0], cidx, sem).wait()
                pltpu.make_async_copy(
                    src_hbm.at[pl.ds(0, CHUNK)], buf, sem).wait()

            load(0, cidx0, buf0, sem0)

            def pair(g, _):
                wait(cidx0, buf0, sem0)
                load(2 * g + 1, cidx1, buf1, sem1)
                pltpu.sync_copy(buf0, acc_a.at[cidx0], add=True)
                wait(cidx1, buf1, sem1)

                @pl.when(g < NPAIR - 1)
                def _():
                    load(2 * g + 2, cidx0, buf0, sem0)

                pltpu.sync_copy(buf1, acc_b.at[cidx1], add=True)
                return 0

            lax.fori_loop(0, NPAIR, pair, 0)

        @pl.when(c == 0)
        def _():
            run(cl_hbm)

        @pl.when(c == 1)
        def _():
            run(cr_hbm)

        plsc.subcore_barrier()

        @pl.when(s < NUM_TILES - 1)
        def _():
            pltpu.sync_copy(acc_a.at[pl.ds(s * STRIPE, STRIPE)],
                            out_hbm.at[c, 0, pl.ds(s * STRIPE, STRIPE)])
            pltpu.sync_copy(acc_b.at[pl.ds(s * STRIPE, STRIPE)],
                            out_hbm.at[c, 1, pl.ds(s * STRIPE, STRIPE)])

        @pl.when(s == NUM_TILES - 1)
        def _():
            pltpu.sync_copy(acc_a.at[pl.ds(s * STRIPE, LAST_STRIPE)],
                            out_hbm.at[c, 0, pl.ds(s * STRIPE, LAST_STRIPE)])
            pltpu.sync_copy(acc_b.at[pl.ds(s * STRIPE, LAST_STRIPE)],
                            out_hbm.at[c, 1, pl.ds(s * STRIPE, LAST_STRIPE)])

    return body(contrib_l, contrib_r, idx, zeros)


def _bn_gelu_tc(la, lb, ra, rb, gamma, beta):
    def body(la_ref, lb_ref, ra_ref, rb_ref, g_ref, b_ref, o_ref):
        vl = la_ref[...].astype(jnp.float32) + lb_ref[...].astype(jnp.float32)
        vr = ra_ref[...].astype(jnp.float32) + rb_ref[...].astype(jnp.float32)
        v = jnp.concatenate([vl, vr], axis=1)
        mean = jnp.mean(v, axis=0, keepdims=True)
        var = jnp.mean((v - mean) ** 2, axis=0, keepdims=True)
        xhat = (v - mean) * lax.rsqrt(var + BN_EPS)
        y = xhat * g_ref[...] + b_ref[...]
        o_ref[...] = 0.5 * y * (1.0 + lax.erf(y * 0.7071067811865476))

    return pl.pallas_call(
        body,
        out_shape=jax.ShapeDtypeStruct((N, C_OUT), jnp.float32),
    )(la, lb, ra, rb, gamma.reshape(1, C_OUT), beta.reshape(1, C_OUT))


def kernel(data, neigh, depth, weight, gamma, beta):
    del depth
    contrib_l, contrib_r = _matmul_tc(data, weight)

    idx_flat = neigh.T.reshape(-1)
    idx = jnp.concatenate(
        [idx_flat, jnp.full((E_PAD - E,), DUMP, jnp.int32)]
    ).reshape(NUM_TILES, NCHUNK, CHUNK)

    zeros = jnp.zeros((STRIPE, C_HALF), jnp.bfloat16)
    p = _scatter_sc(contrib_l, contrib_r, idx, zeros)
    return _bn_gelu_tc(p[0, 0, :N], p[0, 1, :N], p[1, 0, :N], p[1, 1, :N],
                       gamma, beta)


# speedup vs baseline: 2.2153x; 2.2153x over previous
import functools

import jax
import jax.numpy as jnp
from jax import lax
from jax.experimental import pallas as pl
from jax.experimental.pallas import tpu as pltpu
from jax.experimental.pallas import tpu_sc as plsc

N = 10000
C_IN = 256
C_OUT = 256
C_HALF = 128
K = 27
BN_EPS = 1e-5

E = N * K
NUM_TILES = 16
CHUNK = 128
NCHUNK = 132
NPAIR = NCHUNK // 2
E_TILE = NCHUNK * CHUNK
E_PAD = NUM_TILES * E_TILE
DUMP = N
ACC_ROWS = 10008
STRIPE = 632
LAST_STRIPE = ACC_ROWS - (NUM_TILES - 1) * STRIPE


def _matmul_tc(data, weight):
    def body(d_ref, w_ref, l_ref, r_ref):
        res = jnp.dot(d_ref[...], w_ref[0],
                      preferred_element_type=jnp.float32)
        l_ref[...] = res[:, :C_HALF]
        r_ref[...] = res[:, C_HALF:]

    return pl.pallas_call(
        body,
        grid=(K,),
        in_specs=[
            pl.BlockSpec((N, C_IN), lambda k: (0, 0)),
            pl.BlockSpec((1, C_IN, C_OUT), lambda k: (k, 0, 0)),
        ],
        out_specs=[
            pl.BlockSpec((N, C_HALF), lambda k: (k, 0)),
            pl.BlockSpec((N, C_HALF), lambda k: (k, 0)),
        ],
        out_shape=[
            jax.ShapeDtypeStruct((E_PAD, C_HALF), jnp.float32),
            jax.ShapeDtypeStruct((E_PAD, C_HALF), jnp.float32),
        ],
    )(data.astype(jnp.bfloat16), weight.astype(jnp.bfloat16))


def _scatter_sc(contrib_l, contrib_r, idx, zeros):
    mesh = plsc.VectorSubcoreMesh(core_axis_name="c", subcore_axis_name="s")

    @functools.partial(
        pl.kernel,
        out_type=jax.ShapeDtypeStruct((2, ACC_ROWS, C_HALF), jnp.float32),
        mesh=mesh,
        compiler_params=pltpu.CompilerParams(use_tc_tiling_on_sc=False),
        scratch_types=[
            pltpu.VMEM((CHUNK,), jnp.int32),
            pltpu.VMEM((CHUNK,), jnp.int32),
            pltpu.VMEM((CHUNK, C_HALF), jnp.float32),
            pltpu.VMEM((CHUNK, C_HALF), jnp.float32),
            pltpu.VMEM_SHARED((ACC_ROWS, C_HALF), jnp.float32),
            pltpu.SemaphoreType.DMA,
            pltpu.SemaphoreType.DMA,
        ],
    )
    def body(cl_hbm, cr_hbm, idx_hbm, zeros_hbm, out_hbm,
             cidx0, cidx1, buf0, buf1, acc_a, sem0, sem1):
        c = lax.axis_index("c")
        s = lax.axis_index("s")
        base = s * E_TILE

        @pl.when(s < NUM_TILES - 1)
        def _():
            pltpu.sync_copy(zeros_hbm, acc_a.at[pl.ds(s * STRIPE, STRIPE)])

        @pl.when(s == NUM_TILES - 1)
        def _():
            pltpu.sync_copy(zeros_hbm.at[pl.ds(0, LAST_STRIPE)],
                            acc_a.at[pl.ds(s * STRIPE, LAST_STRIPE)])

        plsc.subcore_barrier()

        def run(src_hbm):
            def load(it, cidx, buf, sem):
                pltpu.async_copy(idx_hbm.at[s, it], cidx, sem)
                pltpu.async_copy(
                    src_hbm.at[pl.ds(base + it * CHUNK, CHUNK)], buf, sem)

            def wait(cidx, buf, sem):
                pltpu.make_async_copy(idx_hbm.at[s, 0], cidx, sem).wait()
                pltpu.make_async_copy(
                    src_hbm.at[pl.ds(0, CHUNK)], buf, sem).wait()

            load(0, cidx0, buf0, sem0)

            def pair(g, _):
                wait(cidx0, buf0, sem0)
                load(2 * g + 1, cidx1, buf1, sem1)
                pltpu.sync_copy(buf0, acc_a.at[cidx0], add=True)
                wait(cidx1, buf1, sem1)

                @pl.when(g < NPAIR - 1)
                def _():
                    load(2 * g + 2, cidx0, buf0, sem0)

                pltpu.sync_copy(buf1, acc_a.at[cidx1], add=True)
                return 0

            lax.fori_loop(0, NPAIR, pair, 0)

        @pl.when(c == 0)
        def _():
            run(cl_hbm)

        @pl.when(c == 1)
        def _():
            run(cr_hbm)

        plsc.subcore_barrier()

        @pl.when(s < NUM_TILES - 1)
        def _():
            pltpu.sync_copy(acc_a.at[pl.ds(s * STRIPE, STRIPE)],
                            out_hbm.at[c, pl.ds(s * STRIPE, STRIPE)])

        @pl.when(s == NUM_TILES - 1)
        def _():
            pltpu.sync_copy(acc_a.at[pl.ds(s * STRIPE, LAST_STRIPE)],
                            out_hbm.at[c, pl.ds(s * STRIPE, LAST_STRIPE)])

    return body(contrib_l, contrib_r, idx, zeros)


def _bn_gelu_tc(vl_in, vr_in, gamma, beta):
    def body(vl_ref, vr_ref, g_ref, b_ref, o_ref):
        v = jnp.concatenate([vl_ref[...], vr_ref[...]], axis=1)
        mean = jnp.mean(v, axis=0, keepdims=True)
        var = jnp.mean((v - mean) ** 2, axis=0, keepdims=True)
        xhat = (v - mean) * lax.rsqrt(var + BN_EPS)
        y = xhat * g_ref[...] + b_ref[...]
        o_ref[...] = 0.5 * y * (1.0 + lax.erf(y * 0.7071067811865476))

    return pl.pallas_call(
        body,
        out_shape=jax.ShapeDtypeStruct((N, C_OUT), jnp.float32),
    )(vl_in, vr_in, gamma.reshape(1, C_OUT), beta.reshape(1, C_OUT))


def kernel(data, neigh, depth, weight, gamma, beta):
    del depth
    contrib_l, contrib_r = _matmul_tc(data, weight)

    idx_flat = neigh.T.reshape(-1)
    idx = jnp.concatenate(
        [idx_flat, jnp.full((E_PAD - E,), DUMP, jnp.int32)]
    ).reshape(NUM_TILES, NCHUNK, CHUNK)

    zeros = jnp.zeros((STRIPE, C_HALF), jnp.float32)
    p = _scatter_sc(contrib_l, contrib_r, idx, zeros)
    return _bn_gelu_tc(p[0, :N], p[1, :N], gamma, beta)


# speedup vs baseline: 2.3483x; 1.0600x over previous
import functools

import jax
import jax.numpy as jnp
from jax import lax
from jax.experimental import pallas as pl
from jax.experimental.pallas import tpu as pltpu
from jax.experimental.pallas import tpu_sc as plsc

N = 10000
C_IN = 256
C_OUT = 256
C_HALF = 128
K = 27
BN_EPS = 1e-5

E = N * K
NUM_TILES = 16
CHUNK = 64
NCHUNK = 264
NGROUP = NCHUNK // 4
E_TILE = NCHUNK * CHUNK
E_PAD = NUM_TILES * E_TILE
DUMP = N
ACC_ROWS = 10008
STRIPE = 632
LAST_STRIPE = ACC_ROWS - (NUM_TILES - 1) * STRIPE


def _matmul_tc(data, weight):
    def body(d_ref, w_ref, l_ref, r_ref):
        res = jnp.dot(d_ref[...], w_ref[0],
                      preferred_element_type=jnp.float32)
        l_ref[...] = res[:, :C_HALF]
        r_ref[...] = res[:, C_HALF:]

    return pl.pallas_call(
        body,
        grid=(K,),
        in_specs=[
            pl.BlockSpec((N, C_IN), lambda k: (0, 0)),
            pl.BlockSpec((1, C_IN, C_OUT), lambda k: (k, 0, 0)),
        ],
        out_specs=[
            pl.BlockSpec((N, C_HALF), lambda k: (k, 0)),
            pl.BlockSpec((N, C_HALF), lambda k: (k, 0)),
        ],
        out_shape=[
            jax.ShapeDtypeStruct((E_PAD, C_HALF), jnp.float32),
            jax.ShapeDtypeStruct((E_PAD, C_HALF), jnp.float32),
        ],
    )(data.astype(jnp.bfloat16), weight.astype(jnp.bfloat16))


def _scatter_sc(contrib_l, contrib_r, idx, zeros):
    mesh = plsc.VectorSubcoreMesh(core_axis_name="c", subcore_axis_name="s")

    @functools.partial(
        pl.kernel,
        out_type=jax.ShapeDtypeStruct((2, ACC_ROWS, C_HALF), jnp.float32),
        mesh=mesh,
        compiler_params=pltpu.CompilerParams(use_tc_tiling_on_sc=False),
        scratch_types=[
            pltpu.VMEM((CHUNK,), jnp.int32),
            pltpu.VMEM((CHUNK,), jnp.int32),
            pltpu.VMEM((CHUNK,), jnp.int32),
            pltpu.VMEM((CHUNK,), jnp.int32),
            pltpu.VMEM((CHUNK, C_HALF), jnp.float32),
            pltpu.VMEM((CHUNK, C_HALF), jnp.float32),
            pltpu.VMEM((CHUNK, C_HALF), jnp.float32),
            pltpu.VMEM((CHUNK, C_HALF), jnp.float32),
            pltpu.VMEM_SHARED((ACC_ROWS, C_HALF), jnp.float32),
            pltpu.SemaphoreType.DMA,
            pltpu.SemaphoreType.DMA,
            pltpu.SemaphoreType.DMA,
            pltpu.SemaphoreType.DMA,
            pltpu.SemaphoreType.DMA,
            pltpu.SemaphoreType.DMA,
        ],
    )
    def body(cl_hbm, cr_hbm, idx_hbm, zeros_hbm, out_hbm,
             ci0, ci1, ci2, ci3, b0, b1, b2, b3, acc,
             ls0, ls1, ls2, ls3, ss0, ss1):
        cidxs = [ci0, ci1, ci2, ci3]
        bufs = [b0, b1, b2, b3]
        lsems = [ls0, ls1, ls2, ls3]
        ssems = [ss0, ss1]
        c = lax.axis_index("c")
        s = lax.axis_index("s")
        base = s * E_TILE

        @pl.when(s < NUM_TILES - 1)
        def _():
            pltpu.sync_copy(zeros_hbm, acc.at[pl.ds(s * STRIPE, STRIPE)])

        @pl.when(s == NUM_TILES - 1)
        def _():
            pltpu.sync_copy(zeros_hbm.at[pl.ds(0, LAST_STRIPE)],
                            acc.at[pl.ds(s * STRIPE, LAST_STRIPE)])

        plsc.subcore_barrier()

        def run(src_hbm):
            def load(it, b):
                pltpu.async_copy(idx_hbm.at[s, it], cidxs[b], lsems[b])
                pltpu.async_copy(
                    src_hbm.at[pl.ds(base + it * CHUNK, CHUNK)],
                    bufs[b], lsems[b])

            def wait_load(b):
                pltpu.make_async_copy(
                    idx_hbm.at[s, 0], cidxs[b], lsems[b]).wait()
                pltpu.make_async_copy(
                    src_hbm.at[pl.ds(0, CHUNK)], bufs[b], lsems[b]).wait()

            def scatter_start(b, p):
                pltpu.async_copy(bufs[b], acc.at[cidxs[b]], ssems[p])

            def scatter_wait(b, p):
                pltpu.make_async_copy(
                    bufs[b], acc.at[cidxs[b]], ssems[p]).wait()

            load(0, 0)
            load(1, 1)

            def group(g, _):
                for j in range(4):
                    t = 4 * g + j
                    wait_load(j)

                    @pl.when(t >= 2)
                    def _():
                        scatter_wait((j + 2) % 4, j % 2)

                    scatter_start(j, j % 2)

                    @pl.when(t + 2 < NCHUNK)
                    def _():
                        load(t + 2, (j + 2) % 4)
                return 0

            lax.fori_loop(0, NGROUP, group, 0)
            scatter_wait(2, 0)
            scatter_wait(3, 1)

        @pl.when(c == 0)
        def _():
            run(cl_hbm)

        @pl.when(c == 1)
        def _():
            run(cr_hbm)

        plsc.subcore_barrier()

        @pl.when(s < NUM_TILES - 1)
        def _():
            pltpu.sync_copy(acc.at[pl.ds(s * STRIPE, STRIPE)],
                            out_hbm.at[c, pl.ds(s * STRIPE, STRIPE)])

        @pl.when(s == NUM_TILES - 1)
        def _():
            pltpu.sync_copy(acc.at[pl.ds(s * STRIPE, LAST_STRIPE)],
                            out_hbm.at[c, pl.ds(s * STRIPE, LAST_STRIPE)])

    return body(contrib_l, contrib_r, idx, zeros)


def _bn_gelu_tc(vl_in, vr_in, gamma, beta):
    def body(vl_ref, vr_ref, g_ref, b_ref, o_ref):
        v = jnp.concatenate([vl_ref[...], vr_ref[...]], axis=1)
        mean = jnp.mean(v, axis=0, keepdims=True)
        var = jnp.mean((v - mean) ** 2, axis=0, keepdims=True)
        xhat = (v - mean) * lax.rsqrt(var + BN_EPS)
        y = xhat * g_ref[...] + b_ref[...]
        o_ref[...] = 0.5 * y * (1.0 + lax.erf(y * 0.7071067811865476))

    return pl.pallas_call(
        body,
        out_shape=jax.ShapeDtypeStruct((N, C_OUT), jnp.float32),
    )(vl_in, vr_in, gamma.reshape(1, C_OUT), beta.reshape(1, C_OUT))


def kernel(data, neigh, depth, weight, gamma, beta):
    del depth
    contrib_l, contrib_r = _matmul_tc(data, weight)

    idx_flat = neigh.T.reshape(-1)
    idx = jnp.concatenate(
        [idx_flat, jnp.full((E_PAD - E,), DUMP, jnp.int32)]
    ).reshape(NUM_TILES, NCHUNK, CHUNK)

    zeros = jnp.zeros((STRIPE, C_HALF), jnp.float32)
    p = _scatter_sc(contrib_l, contrib_r, idx, zeros)
    return _bn_gelu_tc(p[0, :N], p[1, :N], gamma, beta)
